# Initial kernel scaffold; baseline (speedup 1.0000x reference)
import jax
import jax.numpy as jnp
from jax.experimental import pallas as pl


def kernel(boxes, scores):
    raise NotImplementedError("write your pallas kernel here")


# speedup vs baseline: 26.4194x; 26.4194x over previous
import numpy as np
import jax
import jax.numpy as jnp
from jax import lax
from jax.experimental import pallas as pl

_N = 5000
_R = 8
_C = 640
_NP = _R * _C
_IOU_T = 0.1
_TWO_PI = 2.0 * np.pi
_PI = np.pi
_NEG = -3.0e38


def _matcher_kernel(x_ref, y_ref, z_ref, dx_ref, dy_ref, dz_ref, r_ref, s_ref,
                    out_ref):
    idx = (lax.broadcasted_iota(jnp.int32, (_R, _C), 0) * _C
           + lax.broadcasted_iota(jnp.int32, (_R, _C), 1))
    valid = idx < _N

    x = x_ref[...]
    y = y_ref[...]
    z = z_ref[...]
    dx = dx_ref[...]
    dy = dy_ref[...]
    dz = dz_ref[...]
    r = r_ref[...]
    s = s_ref[...]

    rr = r - jnp.floor(r / _TWO_PI + 0.5) * _TWO_PI
    cr = jnp.cos(rr)
    sr = jnp.sin(rr)
    ca = jnp.abs(cr)
    sa = jnp.abs(sr)
    hx = 0.5 * (dx * ca + dy * sa)
    hy = 0.5 * (dx * sa + dy * ca)
    x1 = x - hx
    x2 = x + hx
    y1 = y - hy
    y2 = y + hy
    z1 = z - 0.5 * dz
    z2 = z + 0.5 * dz
    vol = (x2 - x1) * (y2 - y1) * (z2 - z1)

    def _find_next(cov):
        return jnp.min(jnp.where(cov == 1, _NP, idx))

    def _cond(st):
        _, _, _, nxt = st
        return nxt < _N

    def _body(st):
        cov, seg, cnum, nxt = st
        mm = idx == nxt

        def ext(a):
            return jnp.sum(jnp.where(mm, a, 0.0))

        xx1 = ext(x1)
        xx2 = ext(x2)
        yy1 = ext(y1)
        yy2 = ext(y2)
        zz1 = ext(z1)
        zz2 = ext(z2)
        vv = ext(vol)
        ix = jnp.maximum(jnp.minimum(x2, xx2) - jnp.maximum(x1, xx1), 0.0)
        iy = jnp.maximum(jnp.minimum(y2, yy2) - jnp.maximum(y1, yy1), 0.0)
        iz = jnp.maximum(jnp.minimum(z2, zz2) - jnp.maximum(z1, zz1), 0.0)
        inter = ix * iy * iz
        union = jnp.maximum(vol + vv - inter, 1e-6)
        mrow = jnp.logical_and(inter / union > _IOU_T, valid)
        cov = jnp.where(mrow, 1, cov)
        seg = jnp.where(mrow, cnum, seg)
        return cov, seg, cnum + 1, _find_next(cov)

    cov0 = jnp.where(valid, 0, 1)
    st0 = (cov0, jnp.zeros((_R, _C), jnp.int32), jnp.int32(0), jnp.int32(0))
    _, seg, nseg, _ = lax.while_loop(_cond, _body, st0)

    out_ref[...] = jnp.zeros((_NP, 16), jnp.float32)
    lane = lax.broadcasted_iota(jnp.int32, (1, 16), 1)

    def _fcond(c):
        return c < nseg

    def _fbody(c):
        m = jnp.logical_and(seg == c, valid)
        cnt = jnp.sum(jnp.where(m, 1.0, 0.0))
        sum_s = jnp.sum(jnp.where(m, s, 0.0))
        max_s = jnp.max(jnp.where(m, s, _NEG))
        eqm = jnp.logical_and(m, s >= max_s)
        ref_idx = jnp.min(jnp.where(eqm, idx, _NP))
        ref_dir = jnp.sum(jnp.where(idx == ref_idx, rr, 0.0))
        diff = jnp.abs(rr - ref_dir)
        diff = jnp.where(diff > _PI, _TWO_PI - diff, diff)
        m_a = diff > (_PI / 2.0)
        s_lt = jnp.sum(jnp.where(jnp.logical_and(m, m_a), s, 0.0))
        s_set = jnp.sum(jnp.where(jnp.logical_and(m, jnp.logical_not(m_a)),
                                  s, 0.0))
        flip_a = s_lt <= s_set
        sgn = jnp.where(jnp.logical_xor(m_a, flip_a), 1.0, -1.0)
        w = jnp.where(m, s / jnp.maximum(sum_s, 1e-12), 0.0)
        sint = jnp.sum(sgn * sr * w)
        cost = jnp.sum(sgn * cr * w)
        theta = jnp.arctan2(sint, cost)
        vals = [jnp.sum(x * w), jnp.sum(y * w), jnp.sum(z * w),
                jnp.sum(dx * w), jnp.sum(dy * w), jnp.sum(dz * w),
                theta, max_s, cnt]
        row = jnp.zeros((1, 16), jnp.float32)
        for k, v in enumerate(vals):
            row = jnp.where(lane == k, v, row)
        row = jnp.where(cnt > 0.0, row, jnp.zeros((1, 16), jnp.float32))
        out_ref[pl.ds(c, 1), :] = row
        return c + 1

    lax.while_loop(_fcond, _fbody, jnp.int32(0))


def _pad2d(v):
    return jnp.pad(v, (0, _NP - _N)).reshape(_R, _C)


@jax.jit
def kernel(boxes, scores):
    cols = [_pad2d(boxes[:, k]) for k in range(7)]
    sv = _pad2d(scores)
    out = pl.pallas_call(
        _matcher_kernel,
        out_shape=jax.ShapeDtypeStruct((_NP, 16), jnp.float32),
    )(*cols, sv)
    boxes_fused = out[:_N, 0:7]
    scores_fused = out[:_N, 7]
    counts = out[:_N, 8]
    return boxes_fused, scores_fused, counts


# speedup vs baseline: 73.1667x; 2.7694x over previous
import numpy as np
import jax
import jax.numpy as jnp
from jax import lax
from jax.experimental import pallas as pl
from jax.experimental.pallas import tpu as pltpu

_N = 5000
_R = 8
_C = 640
_NP = _R * _C
_IOU_T = 0.1
_TWO_PI = 2.0 * np.pi
_PI = np.pi
_NEG = -3.0e38


_SB = 128


def _matcher_kernel(x_ref, y_ref, z_ref, dx_ref, dy_ref, dz_ref, r_ref, s_ref,
                    r_row_ref, s_row_ref, v8_ref, out_ref, segrow_ref):
    idx = (lax.broadcasted_iota(jnp.int32, (_R, _C), 0) * _C
           + lax.broadcasted_iota(jnp.int32, (_R, _C), 1))
    valid = idx < _N

    x = x_ref[...]
    y = y_ref[...]
    z = z_ref[...]
    dx = dx_ref[...]
    dy = dy_ref[...]
    dz = dz_ref[...]
    r = r_ref[...]
    s = s_ref[...]

    rr = r - jnp.floor(r / _TWO_PI + 0.5) * _TWO_PI
    cr = jnp.cos(rr)
    sr = jnp.sin(rr)
    ca = jnp.abs(cr)
    sa = jnp.abs(sr)
    hx = 0.5 * (dx * ca + dy * sa)
    hy = 0.5 * (dx * sa + dy * ca)
    x1 = x - hx
    x2 = x + hx
    y1 = y - hy
    y2 = y + hy
    z1 = z - 0.5 * dz
    z2 = z + 0.5 * dz
    vol = (x2 - x1) * (y2 - y1) * (z2 - z1)

    def _find_next(cov):
        return jnp.min(jnp.where(cov == 1, _NP, idx))

    def _cond(st):
        _, _, _, nxt = st
        return nxt < _N

    def _body(st):
        cov, seg, cnum, nxt = st
        mm = idx == nxt

        def ext(a):
            return jnp.sum(jnp.where(mm, a, 0.0))

        xx1 = ext(x1)
        xx2 = ext(x2)
        yy1 = ext(y1)
        yy2 = ext(y2)
        zz1 = ext(z1)
        zz2 = ext(z2)
        vv = ext(vol)
        ix = jnp.maximum(jnp.minimum(x2, xx2) - jnp.maximum(x1, xx1), 0.0)
        iy = jnp.maximum(jnp.minimum(y2, yy2) - jnp.maximum(y1, yy1), 0.0)
        iz = jnp.maximum(jnp.minimum(z2, zz2) - jnp.maximum(z1, zz1), 0.0)
        inter = ix * iy * iz
        union = jnp.maximum(vol + vv - inter, 1e-6)
        mrow = jnp.logical_and(inter / union > _IOU_T, valid)
        cov = jnp.where(mrow, 1, cov)
        seg = jnp.where(mrow, cnum, seg)
        return cov, seg, cnum + 1, _find_next(cov)

    cov0 = jnp.where(valid, 0, 1)
    st0 = (cov0, jnp.zeros((_R, _C), jnp.int32), jnp.int32(0), jnp.int32(0))
    _, seg, nseg, _ = lax.while_loop(_cond, _body, st0)

    for rrow in range(_R):
        segrow_ref[0:1, rrow * _C:(rrow + 1) * _C] = seg[rrow:rrow + 1, :]
    seg_row = segrow_ref[...]

    idx_row = lax.broadcasted_iota(jnp.int32, (1, _NP), 1)
    valid_row = idx_row < _N
    r_row = r_row_ref[...]
    s_row = s_row_ref[...]
    rr_row = r_row - jnp.floor(r_row / _TWO_PI + 0.5) * _TWO_PI
    sr_row = jnp.sin(rr_row)
    cr_row = jnp.cos(rr_row)
    v8 = v8_ref[...]
    lane16 = lax.broadcasted_iota(jnp.int32, (_SB, 16), 1)

    out_ref[...] = jnp.zeros((_NP, 16), jnp.float32)

    def _dot(a, b):
        return jax.lax.dot_general(a, b, (((1,), (0,)), ((), ())),
                                   preferred_element_type=jnp.float32)

    def _fcond(cb):
        return cb * _SB < nseg

    def _fbody(cb):
        base = cb * _SB
        cid = base + lax.broadcasted_iota(jnp.int32, (_SB, 1), 0)
        am = jnp.logical_and(seg_row == cid, valid_row)
        af = jnp.where(am, 1.0, 0.0)
        s1 = _dot(af, v8)
        cnt = s1[:, 0:1]
        sum_s = s1[:, 1:2]
        max_s = jnp.max(jnp.where(am, s_row, _NEG), axis=1, keepdims=True)
        eqm = jnp.logical_and(am, s_row >= max_s)
        ridx = jnp.min(jnp.where(eqm, idx_row, _NP), axis=1, keepdims=True)
        ref_dir = jnp.sum(jnp.where(idx_row == ridx, rr_row, 0.0),
                          axis=1, keepdims=True)
        diff = jnp.abs(rr_row - ref_dir)
        diff = jnp.where(diff > _PI, _TWO_PI - diff, diff)
        m_a = diff > (_PI / 2.0)
        s_lt = jnp.sum(jnp.where(jnp.logical_and(am, m_a), s_row, 0.0),
                       axis=1, keepdims=True)
        s_set = jnp.sum(
            jnp.where(jnp.logical_and(am, jnp.logical_not(m_a)), s_row, 0.0),
            axis=1, keepdims=True)
        flip_a = s_lt <= s_set
        sgn = jnp.where(jnp.logical_xor(m_a, flip_a), 1.0, -1.0)
        w = jnp.where(am, s_row, 0.0) / jnp.maximum(sum_s, 1e-12)
        cdim = _dot(w, v8)
        wsgn = w * sgn
        sint = jnp.sum(wsgn * sr_row, axis=1, keepdims=True)
        cost = jnp.sum(wsgn * cr_row, axis=1, keepdims=True)
        theta = jnp.arctan2(sint, cost)
        vals = [cdim[:, 2:3], cdim[:, 3:4], cdim[:, 4:5], cdim[:, 5:6],
                cdim[:, 6:7], cdim[:, 7:8], theta, max_s, cnt]
        rows = jnp.zeros((_SB, 16), jnp.float32)
        for k, v in enumerate(vals):
            rows = jnp.where(lane16 == k, v, rows)
        rows = jnp.where(cnt > 0.0, rows, jnp.zeros((_SB, 16), jnp.float32))
        out_ref[pl.ds(base, _SB), :] = rows
        return cb + 1

    lax.while_loop(_fcond, _fbody, jnp.int32(0))


def _pad2d(v):
    return jnp.pad(v, (0, _NP - _N)).reshape(_R, _C)


@jax.jit
def kernel(boxes, scores):
    cols = [_pad2d(boxes[:, k]) for k in range(7)]
    sv = _pad2d(scores)
    r_row = jnp.pad(boxes[:, 6], (0, _NP - _N)).reshape(1, _NP)
    s_row = jnp.pad(scores, (0, _NP - _N)).reshape(1, _NP)
    ones = jnp.ones((_N,), jnp.float32)
    v8 = jnp.pad(
        jnp.stack([ones, scores, boxes[:, 0], boxes[:, 1], boxes[:, 2],
                   boxes[:, 3], boxes[:, 4], boxes[:, 5]], axis=1),
        ((0, _NP - _N), (0, 0)))
    out = pl.pallas_call(
        _matcher_kernel,
        out_shape=jax.ShapeDtypeStruct((_NP, 16), jnp.float32),
        scratch_shapes=[pltpu.VMEM((1, _NP), jnp.int32)],
    )(*cols, sv, r_row, s_row, v8)
    boxes_fused = out[:_N, 0:7]
    scores_fused = out[:_N, 7]
    counts = out[:_N, 8]
    return boxes_fused, scores_fused, counts
